# baseline (device time: 105393 ns/iter reference)
import jax
import jax.numpy as jnp
from jax import lax
from jax.experimental import pallas as pl
from jax.experimental.pallas import tpu as pltpu

N_DEV = 8
E_LOC = 4
N_EXP = N_DEV * E_LOC
CAP = 102
N_TOK = 512
D_IN = 256
D_OUT = 512


def kernel(x, router_W, route_idx, expert_W):
    del router_W

    def body(x_ref, idx_ref, w_ref, out_ref, w_all, cnt_all,
             send_w, recv_w, send_c, recv_c):
        my = lax.axis_index("i")
        left = lax.rem(my - 1 + N_DEV, N_DEV)
        right = lax.rem(my + 1, N_DEV)

        bar = pltpu.get_barrier_semaphore()
        pl.semaphore_signal(bar, inc=1, device_id=(left,),
                            device_id_type=pl.DeviceIdType.MESH)
        pl.semaphore_signal(bar, inc=1, device_id=(right,),
                            device_id_type=pl.DeviceIdType.MESH)
        pl.semaphore_wait(bar, 2)

        idx = idx_ref[:, :]
        e_iota = lax.broadcasted_iota(jnp.int32, (N_TOK, N_EXP), 1)
        onehot = idx == e_iota

        counts = jnp.sum(onehot.astype(jnp.int32), axis=0, keepdims=True)
        cnt_all[pl.ds(my, 1), :] = counts
        w_all[pl.ds(my * E_LOC, E_LOC), :, :] = w_ref[:, :, :]

        for h in range(N_DEV - 1):
            org = lax.rem(my - h + N_DEV, N_DEV)
            rdma_w = pltpu.make_async_remote_copy(
                src_ref=w_all.at[pl.ds(org * E_LOC, E_LOC)],
                dst_ref=w_all.at[pl.ds(org * E_LOC, E_LOC)],
                send_sem=send_w.at[h],
                recv_sem=recv_w.at[h],
                device_id=(right,),
                device_id_type=pl.DeviceIdType.MESH,
            )
            rdma_c = pltpu.make_async_remote_copy(
                src_ref=cnt_all.at[pl.ds(org, 1)],
                dst_ref=cnt_all.at[pl.ds(org, 1)],
                send_sem=send_c.at[h],
                recv_sem=recv_c.at[h],
                device_id=(right,),
                device_id_type=pl.DeviceIdType.MESH,
            )
            rdma_w.start()
            rdma_c.start()
            rdma_w.wait()
            rdma_c.wait()

        dev_iota = lax.broadcasted_iota(jnp.int32, (N_DEV, N_EXP), 0)
        prior = jnp.sum(jnp.where(dev_iota < my, cnt_all[:, :], 0),
                        axis=0, keepdims=True)
        prior_pt = jnp.sum(jnp.where(onehot, prior, 0),
                           axis=1, keepdims=True)

        t_row = lax.broadcasted_iota(jnp.int32, (N_TOK, N_TOK), 0)
        t_col = lax.broadcasted_iota(jnp.int32, (N_TOK, N_TOK), 1)
        tri = (t_col < t_row).astype(jnp.bfloat16)
        excl = jnp.dot(tri, onehot.astype(jnp.bfloat16),
                       preferred_element_type=jnp.float32)
        rank_pt = jnp.sum(jnp.where(onehot, excl, 0.0),
                          axis=1, keepdims=True).astype(jnp.int32)
        keep = (prior_pt + rank_pt) < CAP

        xv = x_ref[:, :]
        acc = jnp.zeros((N_TOK, D_OUT), jnp.float32)
        for e in range(N_EXP):
            sel = jnp.logical_and(onehot[:, e:e + 1], keep)
            gate = sel.astype(jnp.float32)
            acc = acc + gate * jnp.dot(xv, w_all[e],
                                       preferred_element_type=jnp.float32)
        out_ref[:, :] = acc

    return pl.pallas_call(
        body,
        out_shape=jax.ShapeDtypeStruct((N_TOK, D_OUT), jnp.float32),
        in_specs=[
            pl.BlockSpec(memory_space=pltpu.VMEM),
            pl.BlockSpec(memory_space=pltpu.VMEM),
            pl.BlockSpec(memory_space=pltpu.VMEM),
        ],
        out_specs=pl.BlockSpec(memory_space=pltpu.VMEM),
        scratch_shapes=[
            pltpu.VMEM((N_EXP, D_IN, D_OUT), jnp.bfloat16),
            pltpu.VMEM((N_DEV, N_EXP), jnp.int32),
            pltpu.SemaphoreType.DMA((N_DEV - 1,)),
            pltpu.SemaphoreType.DMA((N_DEV - 1,)),
            pltpu.SemaphoreType.DMA((N_DEV - 1,)),
            pltpu.SemaphoreType.DMA((N_DEV - 1,)),
        ],
        compiler_params=pltpu.CompilerParams(collective_id=0),
    )(x.astype(jnp.bfloat16), route_idx, expert_W.astype(jnp.bfloat16))


# device time: 82330 ns/iter; 1.2801x vs baseline; 1.2801x over previous
import jax
import jax.numpy as jnp
from jax import lax
from jax.experimental import pallas as pl
from jax.experimental.pallas import tpu as pltpu

N_DEV = 8
E_LOC = 4
N_EXP = N_DEV * E_LOC
CAP = 102
N_TOK = 512
D_IN = 256
D_OUT = 512


def kernel(x, router_W, route_idx, expert_W):
    del router_W

    def body(x_ref, idx_ref, w_ref, out_ref, w_all, cnt_all,
             send_w, recv_w, send_c, recv_c):
        my = lax.axis_index("i")

        bar = pltpu.get_barrier_semaphore()
        for k in range(1, N_DEV):
            p = lax.rem(my + k, N_DEV)
            pl.semaphore_signal(bar, inc=1, device_id=(p,),
                                device_id_type=pl.DeviceIdType.MESH)
        pl.semaphore_wait(bar, N_DEV - 1)

        idx = idx_ref[:, :]
        e_iota = lax.broadcasted_iota(jnp.int32, (N_TOK, N_EXP), 1)
        onehot = idx == e_iota
        counts = jnp.sum(onehot.astype(jnp.int32), axis=0, keepdims=True)
        cnt_all[pl.ds(my, 1), :] = counts

        sends = []
        for k in range(1, N_DEV):
            p = lax.rem(my + k, N_DEV)
            rdma_w = pltpu.make_async_remote_copy(
                src_ref=w_ref,
                dst_ref=w_all.at[pl.ds(my * E_LOC, E_LOC)],
                send_sem=send_w.at[k - 1],
                recv_sem=recv_w.at[my],
                device_id=(p,),
                device_id_type=pl.DeviceIdType.MESH,
            )
            rdma_c = pltpu.make_async_remote_copy(
                src_ref=cnt_all.at[pl.ds(my, 1)],
                dst_ref=cnt_all.at[pl.ds(my, 1)],
                send_sem=send_c.at[k - 1],
                recv_sem=recv_c.at[my],
                device_id=(p,),
                device_id_type=pl.DeviceIdType.MESH,
            )
            rdma_w.start()
            rdma_c.start()
            sends.append((rdma_w, rdma_c))

        xv = x_ref[:, :]

        acc = jnp.zeros((N_TOK, D_OUT), jnp.float32)
        for j in range(E_LOC):
            e = my * E_LOC + j
            gate = (idx == e).astype(jnp.float32)
            acc = acc + gate * jnp.dot(xv, w_ref[j],
                                       preferred_element_type=jnp.float32)

        t_row = lax.broadcasted_iota(jnp.int32, (N_TOK, N_TOK), 0)
        t_col = lax.broadcasted_iota(jnp.int32, (N_TOK, N_TOK), 1)
        tri = (t_col < t_row).astype(jnp.bfloat16)
        excl = jnp.dot(tri, onehot.astype(jnp.bfloat16),
                       preferred_element_type=jnp.float32)
        rank_pt = jnp.sum(jnp.where(onehot, excl, 0.0),
                          axis=1, keepdims=True).astype(jnp.int32)

        for k in range(1, N_DEV):
            s = lax.rem(my + k, N_DEV)
            recv = pltpu.make_async_remote_copy(
                src_ref=w_ref,
                dst_ref=w_all.at[pl.ds(s * E_LOC, E_LOC)],
                send_sem=send_w.at[0],
                recv_sem=recv_w.at[s],
                device_id=(s,),
                device_id_type=pl.DeviceIdType.MESH,
            )
            recv.wait_recv()
            for j in range(E_LOC):
                e = s * E_LOC + j
                gate = (idx == e).astype(jnp.float32)
                w_e = w_all[pl.ds(s * E_LOC + j, 1)].reshape(D_IN, D_OUT)
                acc = acc + gate * jnp.dot(xv, w_e,
                                           preferred_element_type=jnp.float32)

        for k in range(1, N_DEV):
            s = lax.rem(my + k, N_DEV)
            recv = pltpu.make_async_remote_copy(
                src_ref=cnt_all.at[pl.ds(my, 1)],
                dst_ref=cnt_all.at[pl.ds(s, 1)],
                send_sem=send_c.at[0],
                recv_sem=recv_c.at[s],
                device_id=(s,),
                device_id_type=pl.DeviceIdType.MESH,
            )
            recv.wait_recv()

        dev_iota = lax.broadcasted_iota(jnp.int32, (N_DEV, N_EXP), 0)
        prior = jnp.sum(jnp.where(dev_iota < my, cnt_all[:, :], 0),
                        axis=0, keepdims=True)
        prior_pt = jnp.sum(jnp.where(onehot, prior, 0),
                           axis=1, keepdims=True)
        keep = (prior_pt + rank_pt) < CAP

        out_ref[:, :] = keep.astype(jnp.float32) * acc

        for rdma_w, rdma_c in sends:
            rdma_w.wait_send()
            rdma_c.wait_send()

    return pl.pallas_call(
        body,
        out_shape=jax.ShapeDtypeStruct((N_TOK, D_OUT), jnp.float32),
        in_specs=[
            pl.BlockSpec(memory_space=pltpu.VMEM),
            pl.BlockSpec(memory_space=pltpu.VMEM),
            pl.BlockSpec(memory_space=pltpu.VMEM),
        ],
        out_specs=pl.BlockSpec(memory_space=pltpu.VMEM),
        scratch_shapes=[
            pltpu.VMEM((N_EXP, D_IN, D_OUT), jnp.bfloat16),
            pltpu.VMEM((N_DEV, N_EXP), jnp.int32),
            pltpu.SemaphoreType.DMA((N_DEV - 1,)),
            pltpu.SemaphoreType.DMA((N_DEV,)),
            pltpu.SemaphoreType.DMA((N_DEV - 1,)),
            pltpu.SemaphoreType.DMA((N_DEV,)),
        ],
        compiler_params=pltpu.CompilerParams(collective_id=0),
    )(x.astype(jnp.bfloat16), route_idx, expert_W.astype(jnp.bfloat16))
